# Initial kernel scaffold; baseline (speedup 1.0000x reference)
import jax
import jax.numpy as jnp
from jax.experimental import pallas as pl


def kernel(node_type_ids, edge_index, edge_weight, pairs, degrees, node_emb, type_emb, W_in, b_in, W_r1, b_r1, g_r1, be_r1, W_r2, b_r2, g_r2, be_r2, W_out, b_out, W_l1, b_l1, bn_g, bn_b, W_l2, b_l2):
    raise NotImplementedError("write your pallas kernel here")


# speedup vs baseline: 1.3793x; 1.3793x over previous
import functools

import jax
import jax.numpy as jnp
from jax import lax
from jax.experimental import pallas as pl
from jax.experimental.pallas import tpu as pltpu
from jax.experimental.pallas import tpu_sc as plsc

N = 10000
E = 320000
P = 50000
D = 128
T = 16
EPS = 1e-5

NC = 2
NS = 16
NW = NC * NS

NPAD = 10240
RPW = NPAD // NW
CAP = 20000
LCH = 1280
NLCH = E // LCH
GCH = 80

PPAD = 50176
PPW = PPAD // NW
PCH = 112
NPCH = PPW // PCH

_MESH = plsc.VectorSubcoreMesh(core_axis_name="c", subcore_axis_name="s")


def _sc_partition(src, dst, w):
    outs = (
        jax.ShapeDtypeStruct((NW * (CAP + 32),), jnp.int32),
        jax.ShapeDtypeStruct((NW * (CAP + 32),), jnp.int32),
        jax.ShapeDtypeStruct((NW * (CAP + 32),), jnp.float32),
        jax.ShapeDtypeStruct((NW * 128,), jnp.int32),
    )

    @functools.partial(
        pl.kernel,
        mesh=_MESH,
        out_type=outs,
        scratch_types=[
            pltpu.VMEM((LCH,), jnp.int32),
            pltpu.VMEM((LCH,), jnp.int32),
            pltpu.VMEM((LCH,), jnp.float32),
            pltpu.VMEM((CAP + 32,), jnp.int32),
            pltpu.VMEM((CAP + 32,), jnp.int32),
            pltpu.VMEM((CAP + 32,), jnp.float32),
            pltpu.VMEM((128,), jnp.int32),
        ],
    )
    def k(src_hbm, dst_hbm, w_hbm, sl_hbm, dl_hbm, wl_hbm, cnt_hbm,
          sch, dch, wch, slb, dlb, wlb, cbuf):
        cid = lax.axis_index("c")
        sid = lax.axis_index("s")
        wid = sid * NC + cid
        lo = wid * RPW

        zi = jnp.zeros((16,), jnp.int32)
        zf = jnp.zeros((16,), jnp.float32)

        def chunk(c, off):
            pltpu.sync_copy(src_hbm.at[pl.ds(c * LCH, LCH)], sch)
            pltpu.sync_copy(dst_hbm.at[pl.ds(c * LCH, LCH)], dch)
            pltpu.sync_copy(w_hbm.at[pl.ds(c * LCH, LCH)], wch)

            def group(g, o):
                sl = pl.ds(g * 16, 16)
                d16 = dch[sl]
                s16 = sch[sl]
                w16 = wch[sl]
                dloc16 = d16 - lo
                mi = ((dloc16 >> 31) + 1) * (((RPW - 1 - dloc16) >> 31) + 1)
                zi16 = jnp.zeros((16,), jnp.int32)
                zf16 = jnp.zeros((16,), jnp.float32)
                for lane in range(16):
                    ml = mi[lane]
                    oc = jnp.minimum(o, CAP)

                    @pl.when(ml > 0)
                    def _(oc=oc, lane=lane):
                        slb[pl.ds(oc, 16)] = zi16 + s16[lane]
                        dlb[pl.ds(oc, 16)] = zi16 + dloc16[lane]
                        wlb[pl.ds(oc, 16)] = zf16 + w16[lane]

                    o = o + ml
                return jnp.minimum(o, CAP)

            return lax.fori_loop(0, LCH // 16, group, off)

        off = jnp.minimum(lax.fori_loop(0, NLCH, chunk, 0), CAP)
        for t in range(6):
            sl = pl.ds(off + t * 16, 16)
            slb[sl] = zi
            dlb[sl] = zi
            wlb[sl] = zf
        cbuf[pl.ds(0, 16)] = jnp.full((16,), off, jnp.int32)
        lbase = wid * (CAP + 32)
        pltpu.sync_copy(slb, sl_hbm.at[pl.ds(lbase, CAP + 32)])
        pltpu.sync_copy(dlb, dl_hbm.at[pl.ds(lbase, CAP + 32)])
        pltpu.sync_copy(wlb, wl_hbm.at[pl.ds(lbase, CAP + 32)])
        pltpu.sync_copy(cbuf, cnt_hbm.at[pl.ds(wid * 128, 128)])

    return k(src, dst, w)


def _sc_spmm_seq(x, slist, dlist, wlist, cnt):
    @functools.partial(
        pl.kernel,
        mesh=_MESH,
        out_type=jax.ShapeDtypeStruct((NPAD, D), jnp.float32),
        scratch_types=[
            pltpu.VMEM((GCH,), jnp.int32),
            pltpu.VMEM((GCH,), jnp.int32),
            pltpu.VMEM((GCH,), jnp.float32),
            pltpu.VMEM((GCH, D), jnp.float32),
            pltpu.VMEM((RPW, D), jnp.float32),
            pltpu.VMEM((128,), jnp.int32),
            pltpu.SemaphoreType.DMA,
        ],
    )
    def k(x_hbm, sl_hbm, dl_hbm, wl_hbm, cnt_hbm, out_hbm,
          s80, d80, w80, rows, acc, cbuf, sem):
        cid = lax.axis_index("c")
        sid = lax.axis_index("s")
        wid = sid * NC + cid
        lo = wid * RPW

        z16 = jnp.zeros((16,), jnp.float32)

        def zrow(r, carry):
            for c8 in range(D // 16):
                acc[r, pl.ds(c8 * 16, 16)] = z16
            return carry

        lax.fori_loop(0, RPW, zrow, 0)

        pltpu.sync_copy(cnt_hbm.at[pl.ds(wid * 128, 128)], cbuf)
        n = cbuf[pl.ds(0, 16)][0]
        nch = (n + GCH - 1) // GCH

        def chunk(c, carry):
            lbase = wid * (CAP + 32) + c * GCH
            pltpu.sync_copy(sl_hbm.at[pl.ds(lbase, GCH)], s80)
            pltpu.sync_copy(dl_hbm.at[pl.ds(lbase, GCH)], d80)
            pltpu.sync_copy(wl_hbm.at[pl.ds(lbase, GCH)], w80)
            pltpu.async_copy(x_hbm.at[s80], rows, sem).wait()

            def group(g, cc):
                d16 = d80[pl.ds(g * 16, 16)]
                w16 = w80[pl.ds(g * 16, 16)]
                for lane in range(16):
                    e = g * 16 + lane
                    dl = d16[lane]
                    ws = w16[lane]
                    for c8 in range(D // 16):
                        sl = pl.ds(c8 * 16, 16)
                        acc[dl, sl] = acc[dl, sl] + rows[e, sl] * ws
                return cc

            lax.fori_loop(0, GCH // 16, group, 0)
            return carry

        lax.fori_loop(0, nch, chunk, 0)
        pltpu.sync_copy(acc, out_hbm.at[pl.ds(lo, RPW)])

    return k(x, slist, dlist, wlist, cnt)


def _sc_pair_gather(z, psrc, pdst, degrees):
    outs = (
        jax.ShapeDtypeStruct((PPAD, D), jnp.float32),
        jax.ShapeDtypeStruct((PPAD, D), jnp.float32),
        jax.ShapeDtypeStruct((PPAD,), jnp.int32),
        jax.ShapeDtypeStruct((PPAD,), jnp.int32),
    )

    @functools.partial(
        pl.kernel,
        mesh=_MESH,
        out_type=outs,
        scratch_types=[
            pltpu.VMEM((PCH,), jnp.int32),
            pltpu.VMEM((PCH, D), jnp.float32),
            pltpu.VMEM((PCH,), jnp.int32),
            pltpu.SemaphoreType.DMA,
        ],
    )
    def k(z_hbm, ps_hbm, pd_hbm, deg_hbm, sz_hbm, dz_hbm, sd_hbm, dd_hbm,
          idxv, rowbuf, degbuf, sem):
        cid = lax.axis_index("c")
        sid = lax.axis_index("s")
        wid = sid * NC + cid
        base = wid * PPW

        def one_side(p_hbm, zo_hbm, do_hbm):
            def chunk(c, carry):
                off = base + c * PCH
                pltpu.sync_copy(p_hbm.at[pl.ds(off, PCH)], idxv)
                pltpu.async_copy(z_hbm.at[idxv], rowbuf, sem).wait()
                pltpu.sync_copy(rowbuf, zo_hbm.at[pl.ds(off, PCH)])
                pltpu.async_copy(deg_hbm.at[idxv], degbuf, sem).wait()
                pltpu.sync_copy(degbuf, do_hbm.at[pl.ds(off, PCH)])
                return carry

            lax.fori_loop(0, NPCH, chunk, 0)

        one_side(ps_hbm, sz_hbm, sd_hbm)
        one_side(pd_hbm, dz_hbm, dd_hbm)

    return k(z, psrc, pdst, degrees)


def _dot16(a, b):
    return jnp.dot(a.astype(jnp.bfloat16), b.astype(jnp.bfloat16),
                   preferred_element_type=jnp.float32)


def _b16(a):
    return a.astype(jnp.bfloat16).astype(jnp.float32)


def _tc_encode(ids2, node_emb, type_emb):
    EB = 1000

    def body(ids_ref, ne_ref, te_ref, o_ref):
        ids = ids_ref[...]
        te = te_ref[...]
        acc = jnp.zeros((EB, D), jnp.float32)
        for t in range(T):
            acc = jnp.where(ids == t, te[t:t + 1], acc)
        o_ref[...] = ne_ref[...] + acc

    return pl.pallas_call(
        body,
        grid=(N // EB,),
        in_specs=[
            pl.BlockSpec((EB, 1), lambda i: (i, 0)),
            pl.BlockSpec((EB, D), lambda i: (i, 0)),
            pl.BlockSpec((T, D), lambda i: (0, 0)),
        ],
        out_specs=pl.BlockSpec((EB, D), lambda i: (i, 0)),
        out_shape=jax.ShapeDtypeStruct((N, D), jnp.float32),
    )(ids2, node_emb, type_emb)


def _tc_dense_relu(p, W, b):
    def body(p_ref, w_ref, b_ref, o_ref):
        t = p_ref[...][:N]
        h = _dot16(t, w_ref[...])
        o_ref[...] = jnp.maximum(h + b_ref[...], 0.0)

    return pl.pallas_call(
        body, out_shape=jax.ShapeDtypeStruct((N, D), jnp.float32),
    )(p, W, b)


def _tc_dense_ln_res(p, x, W, b, g, be):
    def body(p_ref, x_ref, w_ref, b_ref, g_ref, be_ref, o_ref):
        t = p_ref[...][:N]
        h = _dot16(t, w_ref[...])
        h = h + b_ref[...]
        mu = jnp.mean(h, axis=-1, keepdims=True)
        d = h - mu
        var = jnp.mean(d * d, axis=-1, keepdims=True)
        h = d * lax.rsqrt(var + EPS) * g_ref[...] + be_ref[...]
        o_ref[...] = x_ref[...] + jnp.maximum(h, 0.0)

    return pl.pallas_call(
        body, out_shape=jax.ShapeDtypeStruct((N, D), jnp.float32),
    )(p, x, W, b, g, be)


def _tc_dense_lin(p, W, b):
    def body(p_ref, w_ref, b_ref, o_ref):
        t = p_ref[...][:N]
        o_ref[...] = _dot16(t, w_ref[...]) + b_ref[...]

    return pl.pallas_call(
        body, out_shape=jax.ShapeDtypeStruct((N, D), jnp.float32),
    )(p, W, b)


_SB = 512
_SNB = PPAD // _SB


def _tc_score1(sz, dz, sd, dd, Wa, Wb, Wc, u, v, b1):
    def body(sz_ref, dz_ref, sd_ref, dd_ref, wa_ref, wb_ref, wc_ref, u_ref,
             v_ref, b_ref, h_ref, sum_ref, sq_ref, accs, accq):
        i = pl.program_id(0)
        s = sz_ref[...]
        d = dz_ref[...]
        sl = jnp.log(jnp.maximum(sd_ref[...].astype(jnp.float32), 1.0))
        dl = jnp.log(jnp.maximum(dd_ref[...].astype(jnp.float32), 1.0))
        h = _dot16(s, wa_ref[...])
        h = h + _dot16(d, wb_ref[...])
        h = h + _dot16(s * d, wc_ref[...])
        h = h + _b16(sl) * _b16(u_ref[...]) + _b16(dl) * _b16(v_ref[...])
        h = h + b_ref[...]
        h = jnp.maximum(h, 0.0)
        h_ref[...] = h
        row = i * _SB + lax.broadcasted_iota(jnp.int32, (_SB, 1), 0)
        hm = jnp.where(row < P, h, 0.0)

        @pl.when(i == 0)
        def _():
            accs[...] = jnp.zeros_like(accs)
            accq[...] = jnp.zeros_like(accq)

        accs[...] += jnp.sum(hm, axis=0, keepdims=True)
        accq[...] += jnp.sum(hm * hm, axis=0, keepdims=True)

        @pl.when(i == _SNB - 1)
        def _():
            sum_ref[...] = accs[...]
            sq_ref[...] = accq[...]

    return pl.pallas_call(
        body,
        grid=(_SNB,),
        in_specs=[
            pl.BlockSpec((_SB, D), lambda i: (i, 0)),
            pl.BlockSpec((_SB, D), lambda i: (i, 0)),
            pl.BlockSpec((_SB, 1), lambda i: (i, 0)),
            pl.BlockSpec((_SB, 1), lambda i: (i, 0)),
            pl.BlockSpec((D, D), lambda i: (0, 0)),
            pl.BlockSpec((D, D), lambda i: (0, 0)),
            pl.BlockSpec((D, D), lambda i: (0, 0)),
            pl.BlockSpec((1, D), lambda i: (0, 0)),
            pl.BlockSpec((1, D), lambda i: (0, 0)),
            pl.BlockSpec((1, D), lambda i: (0, 0)),
        ],
        out_specs=[
            pl.BlockSpec((_SB, D), lambda i: (i, 0)),
            pl.BlockSpec((1, D), lambda i: (0, 0)),
            pl.BlockSpec((1, D), lambda i: (0, 0)),
        ],
        out_shape=[
            jax.ShapeDtypeStruct((PPAD, D), jnp.float32),
            jax.ShapeDtypeStruct((1, D), jnp.float32),
            jax.ShapeDtypeStruct((1, D), jnp.float32),
        ],
        scratch_shapes=[
            pltpu.VMEM((1, D), jnp.float32),
            pltpu.VMEM((1, D), jnp.float32),
        ],
    )(sz, dz, sd, dd, Wa, Wb, Wc, u, v, b1)


def _tc_score2(h, hsum, hsq, g, b, W2, b2):
    def body(h_ref, s_ref, q_ref, g_ref, b_ref, w2_ref, b2_ref, o_ref):
        mu = s_ref[...] * (1.0 / P)
        var = q_ref[...] * (1.0 / P) - mu * mu
        hn = (h_ref[...] - mu) * lax.rsqrt(var + EPS) * g_ref[...] + b_ref[...]
        o_ref[...] = _dot16(hn, w2_ref[...]) + b2_ref[...]

    return pl.pallas_call(
        body,
        grid=(_SNB,),
        in_specs=[
            pl.BlockSpec((_SB, D), lambda i: (i, 0)),
            pl.BlockSpec((1, D), lambda i: (0, 0)),
            pl.BlockSpec((1, D), lambda i: (0, 0)),
            pl.BlockSpec((1, D), lambda i: (0, 0)),
            pl.BlockSpec((1, D), lambda i: (0, 0)),
            pl.BlockSpec((D, 1), lambda i: (0, 0)),
            pl.BlockSpec((1, 1), lambda i: (0, 0)),
        ],
        out_specs=pl.BlockSpec((_SB, 1), lambda i: (i, 0)),
        out_shape=jax.ShapeDtypeStruct((PPAD, 1), jnp.float32),
    )(h, hsum, hsq, g, b, W2, b2)


def kernel(node_type_ids, edge_index, edge_weight, pairs, degrees, node_emb,
           type_emb, W_in, b_in, W_r1, b_r1, g_r1, be_r1, W_r2, b_r2, g_r2,
           be_r2, W_out, b_out, W_l1, b_l1, bn_g, bn_b, W_l2, b_l2):
    src = edge_index[1].astype(jnp.int32)
    dst = edge_index[0].astype(jnp.int32)
    ids2 = node_type_ids.astype(jnp.int32).reshape(N, 1)

    x = _tc_encode(ids2, node_emb, type_emb)
    sl, dl, wl, cnt = _sc_partition(src, dst, edge_weight)
    p = _sc_spmm_seq(x, sl, dl, wl, cnt)
    x = _tc_dense_relu(p, W_in, b_in.reshape(1, D))
    p = _sc_spmm_seq(x, sl, dl, wl, cnt)
    x = _tc_dense_ln_res(p, x, W_r1, b_r1.reshape(1, D), g_r1.reshape(1, D),
                         be_r1.reshape(1, D))
    p = _sc_spmm_seq(x, sl, dl, wl, cnt)
    x = _tc_dense_ln_res(p, x, W_r2, b_r2.reshape(1, D), g_r2.reshape(1, D),
                         be_r2.reshape(1, D))
    p = _sc_spmm_seq(x, sl, dl, wl, cnt)
    z = _tc_dense_lin(p, W_out, b_out.reshape(1, D))

    pp = jnp.pad(pairs.astype(jnp.int32), ((0, 0), (0, PPAD - P)))
    sz, dz, sd, dd = _sc_pair_gather(z, pp[0], pp[1],
                                     degrees.astype(jnp.int32))

    h, hsum, hsq = _tc_score1(
        sz, dz, sd.reshape(PPAD, 1), dd.reshape(PPAD, 1),
        W_l1[0:D], W_l1[D:2 * D], W_l1[2 * D:3 * D],
        W_l1[3 * D:3 * D + 1], W_l1[3 * D + 1:3 * D + 2], b_l1.reshape(1, D))
    out2 = _tc_score2(h, hsum, hsq, bn_g.reshape(1, D), bn_b.reshape(1, D),
                      W_l2, b_l2.reshape(1, 1))
    return out2[:P, 0]


# speedup vs baseline: 1.5119x; 1.0961x over previous
import functools

import jax
import jax.numpy as jnp
from jax import lax
from jax.experimental import pallas as pl
from jax.experimental.pallas import tpu as pltpu
from jax.experimental.pallas import tpu_sc as plsc

N = 10000
E = 320000
P = 50000
D = 128
T = 16
EPS = 1e-5

NC = 2
NS = 16
NW = NC * NS

NPAD = 10240
RPW = NPAD // NW
CAP = 20000
LCH = 1280
NLCH = E // LCH
GCH = 96

PPAD = 50176
PPW = PPAD // NW
PCH = 112
NPCH = PPW // PCH

_MESH = plsc.VectorSubcoreMesh(core_axis_name="c", subcore_axis_name="s")


def _sc_partition(src, dst, w):
    outs = (
        jax.ShapeDtypeStruct((NW * (CAP + 32),), jnp.int32),
        jax.ShapeDtypeStruct((NW * (CAP + 32),), jnp.int32),
        jax.ShapeDtypeStruct((NW * (CAP + 32),), jnp.float32),
        jax.ShapeDtypeStruct((NW * 128,), jnp.int32),
    )

    @functools.partial(
        pl.kernel,
        mesh=_MESH,
        out_type=outs,
        scratch_types=[
            pltpu.VMEM((LCH,), jnp.int32),
            pltpu.VMEM((LCH,), jnp.int32),
            pltpu.VMEM((LCH,), jnp.float32),
            pltpu.VMEM((CAP + 32,), jnp.int32),
            pltpu.VMEM((CAP + 32,), jnp.int32),
            pltpu.VMEM((CAP + 32,), jnp.float32),
            pltpu.VMEM((128,), jnp.int32),
        ],
    )
    def k(src_hbm, dst_hbm, w_hbm, sl_hbm, dl_hbm, wl_hbm, cnt_hbm,
          sch, dch, wch, slb, dlb, wlb, cbuf):
        cid = lax.axis_index("c")
        sid = lax.axis_index("s")
        wid = sid * NC + cid
        lo = wid * RPW

        zi = jnp.zeros((16,), jnp.int32)
        zf = jnp.zeros((16,), jnp.float32)

        def chunk(c, off):
            pltpu.sync_copy(src_hbm.at[pl.ds(c * LCH, LCH)], sch)
            pltpu.sync_copy(dst_hbm.at[pl.ds(c * LCH, LCH)], dch)
            pltpu.sync_copy(w_hbm.at[pl.ds(c * LCH, LCH)], wch)

            def group(g, o):
                sl = pl.ds(g * 16, 16)
                d16 = dch[sl]
                s16 = sch[sl]
                w16 = wch[sl]
                dloc16 = d16 - lo
                mi = ((dloc16 >> 31) + 1) * (((RPW - 1 - dloc16) >> 31) + 1)
                zi16 = jnp.zeros((16,), jnp.int32)
                zf16 = jnp.zeros((16,), jnp.float32)
                for lane in range(16):
                    ml = mi[lane]
                    oc = jnp.minimum(o, CAP)

                    @pl.when(ml > 0)
                    def _(oc=oc, lane=lane):
                        slb[pl.ds(oc, 16)] = zi16 + s16[lane]
                        dlb[pl.ds(oc, 16)] = zi16 + dloc16[lane]
                        wlb[pl.ds(oc, 16)] = zf16 + w16[lane]

                    o = o + ml
                return jnp.minimum(o, CAP)

            return lax.fori_loop(0, LCH // 16, group, off)

        off = jnp.minimum(lax.fori_loop(0, NLCH, chunk, 0), CAP)
        for t in range(6):
            sl = pl.ds(off + t * 16, 16)
            slb[sl] = zi
            dlb[sl] = zi
            wlb[sl] = zf
        cbuf[pl.ds(0, 16)] = jnp.full((16,), off, jnp.int32)
        lbase = wid * (CAP + 32)
        pltpu.sync_copy(slb, sl_hbm.at[pl.ds(lbase, CAP + 32)])
        pltpu.sync_copy(dlb, dl_hbm.at[pl.ds(lbase, CAP + 32)])
        pltpu.sync_copy(wlb, wl_hbm.at[pl.ds(lbase, CAP + 32)])
        pltpu.sync_copy(cbuf, cnt_hbm.at[pl.ds(wid * 128, 128)])

    return k(src, dst, w)


def _sc_spmm_seq(x, slist, dlist, wlist, cnt):
    @functools.partial(
        pl.kernel,
        mesh=_MESH,
        out_type=jax.ShapeDtypeStruct((NPAD, D), jnp.float32),
        scratch_types=[
            pltpu.VMEM((CAP + 32,), jnp.int32),
            pltpu.VMEM((CAP + 32,), jnp.float32),
            pltpu.VMEM((GCH,), jnp.int32),
            pltpu.VMEM((GCH, D), jnp.float32),
            pltpu.VMEM((RPW, D), jnp.float32),
            pltpu.VMEM((128,), jnp.int32),
            pltpu.SemaphoreType.DMA,
        ],
    )
    def k(x_hbm, sl_hbm, dl_hbm, wl_hbm, cnt_hbm, out_hbm,
          dlb, wlb, s128, rows, acc, cbuf, sem):
        cid = lax.axis_index("c")
        sid = lax.axis_index("s")
        wid = sid * NC + cid
        lo = wid * RPW

        z16 = jnp.zeros((16,), jnp.float32)

        lbase = wid * (CAP + 32)
        pltpu.sync_copy(dl_hbm.at[pl.ds(lbase, CAP + 32)], dlb)
        pltpu.sync_copy(wl_hbm.at[pl.ds(lbase, CAP + 32)], wlb)
        pltpu.sync_copy(cnt_hbm.at[pl.ds(wid * 128, 128)], cbuf)

        def zrow(r, carry):
            for c8 in range(D // 16):
                acc[r, pl.ds(c8 * 16, 16)] = z16
            return carry

        lax.fori_loop(0, RPW, zrow, 0)

        n = cbuf[pl.ds(0, 16)][0]
        nch = (n + GCH - 1) // GCH

        def chunk(c, carry):
            pltpu.sync_copy(sl_hbm.at[pl.ds(lbase + c * GCH, GCH)], s128)
            pltpu.async_copy(x_hbm.at[s128], rows, sem).wait()

            def group(g, cc):
                d16 = dlb[pl.ds(c * GCH + g * 16, 16)]
                w16 = wlb[pl.ds(c * GCH + g * 16, 16)]
                for lane in range(16):
                    e = g * 16 + lane
                    dl = d16[lane]
                    ws = w16[lane]
                    for c8 in range(D // 16):
                        sl = pl.ds(c8 * 16, 16)
                        acc[dl, sl] = acc[dl, sl] + rows[e, sl] * ws
                return cc

            lax.fori_loop(0, GCH // 16, group, 0)
            return carry

        lax.fori_loop(0, nch, chunk, 0)
        pltpu.sync_copy(acc, out_hbm.at[pl.ds(lo, RPW)])

    return k(x, slist, dlist, wlist, cnt)


def _sc_pair_gather(z, psrc, pdst, degrees):
    outs = (
        jax.ShapeDtypeStruct((PPAD, D), jnp.float32),
        jax.ShapeDtypeStruct((PPAD, D), jnp.float32),
        jax.ShapeDtypeStruct((PPAD,), jnp.int32),
        jax.ShapeDtypeStruct((PPAD,), jnp.int32),
    )

    @functools.partial(
        pl.kernel,
        mesh=_MESH,
        out_type=outs,
        scratch_types=[
            pltpu.VMEM((PCH,), jnp.int32),
            pltpu.VMEM((PCH, D), jnp.float32),
            pltpu.VMEM((PCH,), jnp.int32),
            pltpu.SemaphoreType.DMA,
        ],
    )
    def k(z_hbm, ps_hbm, pd_hbm, deg_hbm, sz_hbm, dz_hbm, sd_hbm, dd_hbm,
          idxv, rowbuf, degbuf, sem):
        cid = lax.axis_index("c")
        sid = lax.axis_index("s")
        wid = sid * NC + cid
        base = wid * PPW

        def one_side(p_hbm, zo_hbm, do_hbm):
            def chunk(c, carry):
                off = base + c * PCH
                pltpu.sync_copy(p_hbm.at[pl.ds(off, PCH)], idxv)
                pltpu.async_copy(z_hbm.at[idxv], rowbuf, sem).wait()
                pltpu.sync_copy(rowbuf, zo_hbm.at[pl.ds(off, PCH)])
                pltpu.async_copy(deg_hbm.at[idxv], degbuf, sem).wait()
                pltpu.sync_copy(degbuf, do_hbm.at[pl.ds(off, PCH)])
                return carry

            lax.fori_loop(0, NPCH, chunk, 0)

        one_side(ps_hbm, sz_hbm, sd_hbm)
        one_side(pd_hbm, dz_hbm, dd_hbm)

    return k(z, psrc, pdst, degrees)


def _dot16(a, b):
    return jnp.dot(a.astype(jnp.bfloat16), b.astype(jnp.bfloat16),
                   preferred_element_type=jnp.float32)


def _b16(a):
    return a.astype(jnp.bfloat16).astype(jnp.float32)


def _tc_encode(ids2, node_emb, type_emb):
    EB = 1000

    def body(ids_ref, ne_ref, te_ref, o_ref):
        ids = ids_ref[...]
        te = te_ref[...]
        acc = jnp.zeros((EB, D), jnp.float32)
        for t in range(T):
            acc = jnp.where(ids == t, te[t:t + 1], acc)
        o_ref[...] = ne_ref[...] + acc

    return pl.pallas_call(
        body,
        grid=(N // EB,),
        in_specs=[
            pl.BlockSpec((EB, 1), lambda i: (i, 0)),
            pl.BlockSpec((EB, D), lambda i: (i, 0)),
            pl.BlockSpec((T, D), lambda i: (0, 0)),
        ],
        out_specs=pl.BlockSpec((EB, D), lambda i: (i, 0)),
        out_shape=jax.ShapeDtypeStruct((N, D), jnp.float32),
    )(ids2, node_emb, type_emb)


def _tc_dense_relu(p, W, b):
    def body(p_ref, w_ref, b_ref, o_ref):
        t = p_ref[...][:N]
        h = _dot16(t, w_ref[...])
        o_ref[...] = jnp.maximum(h + b_ref[...], 0.0)

    return pl.pallas_call(
        body, out_shape=jax.ShapeDtypeStruct((N, D), jnp.float32),
    )(p, W, b)


def _tc_dense_ln_res(p, x, W, b, g, be):
    def body(p_ref, x_ref, w_ref, b_ref, g_ref, be_ref, o_ref):
        t = p_ref[...][:N]
        h = _dot16(t, w_ref[...])
        h = h + b_ref[...]
        mu = jnp.mean(h, axis=-1, keepdims=True)
        d = h - mu
        var = jnp.mean(d * d, axis=-1, keepdims=True)
        h = d * lax.rsqrt(var + EPS) * g_ref[...] + be_ref[...]
        o_ref[...] = x_ref[...] + jnp.maximum(h, 0.0)

    return pl.pallas_call(
        body, out_shape=jax.ShapeDtypeStruct((N, D), jnp.float32),
    )(p, x, W, b, g, be)


def _tc_dense_lin(p, W, b):
    def body(p_ref, w_ref, b_ref, o_ref):
        t = p_ref[...][:N]
        o_ref[...] = _dot16(t, w_ref[...]) + b_ref[...]

    return pl.pallas_call(
        body, out_shape=jax.ShapeDtypeStruct((N, D), jnp.float32),
    )(p, W, b)


_SB = 512
_SNB = PPAD // _SB


def _tc_score1(sz, dz, sd, dd, Wa, Wb, Wc, u, v, b1):
    def body(sz_ref, dz_ref, sd_ref, dd_ref, wa_ref, wb_ref, wc_ref, u_ref,
             v_ref, b_ref, h_ref, sum_ref, sq_ref, accs, accq):
        i = pl.program_id(0)
        s = sz_ref[...]
        d = dz_ref[...]
        sl = jnp.log(jnp.maximum(sd_ref[...].astype(jnp.float32), 1.0))
        dl = jnp.log(jnp.maximum(dd_ref[...].astype(jnp.float32), 1.0))
        h = _dot16(s, wa_ref[...])
        h = h + _dot16(d, wb_ref[...])
        h = h + _dot16(s * d, wc_ref[...])
        h = h + _b16(sl) * _b16(u_ref[...]) + _b16(dl) * _b16(v_ref[...])
        h = h + b_ref[...]
        h = jnp.maximum(h, 0.0)
        h_ref[...] = h
        row = i * _SB + lax.broadcasted_iota(jnp.int32, (_SB, 1), 0)
        hm = jnp.where(row < P, h, 0.0)

        @pl.when(i == 0)
        def _():
            accs[...] = jnp.zeros_like(accs)
            accq[...] = jnp.zeros_like(accq)

        accs[...] += jnp.sum(hm, axis=0, keepdims=True)
        accq[...] += jnp.sum(hm * hm, axis=0, keepdims=True)

        @pl.when(i == _SNB - 1)
        def _():
            sum_ref[...] = accs[...]
            sq_ref[...] = accq[...]

    return pl.pallas_call(
        body,
        grid=(_SNB,),
        in_specs=[
            pl.BlockSpec((_SB, D), lambda i: (i, 0)),
            pl.BlockSpec((_SB, D), lambda i: (i, 0)),
            pl.BlockSpec((_SB, 1), lambda i: (i, 0)),
            pl.BlockSpec((_SB, 1), lambda i: (i, 0)),
            pl.BlockSpec((D, D), lambda i: (0, 0)),
            pl.BlockSpec((D, D), lambda i: (0, 0)),
            pl.BlockSpec((D, D), lambda i: (0, 0)),
            pl.BlockSpec((1, D), lambda i: (0, 0)),
            pl.BlockSpec((1, D), lambda i: (0, 0)),
            pl.BlockSpec((1, D), lambda i: (0, 0)),
        ],
        out_specs=[
            pl.BlockSpec((_SB, D), lambda i: (i, 0)),
            pl.BlockSpec((1, D), lambda i: (0, 0)),
            pl.BlockSpec((1, D), lambda i: (0, 0)),
        ],
        out_shape=[
            jax.ShapeDtypeStruct((PPAD, D), jnp.float32),
            jax.ShapeDtypeStruct((1, D), jnp.float32),
            jax.ShapeDtypeStruct((1, D), jnp.float32),
        ],
        scratch_shapes=[
            pltpu.VMEM((1, D), jnp.float32),
            pltpu.VMEM((1, D), jnp.float32),
        ],
    )(sz, dz, sd, dd, Wa, Wb, Wc, u, v, b1)


def _tc_score2(h, hsum, hsq, g, b, W2, b2):
    def body(h_ref, s_ref, q_ref, g_ref, b_ref, w2_ref, b2_ref, o_ref):
        mu = s_ref[...] * (1.0 / P)
        var = q_ref[...] * (1.0 / P) - mu * mu
        hn = (h_ref[...] - mu) * lax.rsqrt(var + EPS) * g_ref[...] + b_ref[...]
        o_ref[...] = _dot16(hn, w2_ref[...]) + b2_ref[...]

    return pl.pallas_call(
        body,
        grid=(_SNB,),
        in_specs=[
            pl.BlockSpec((_SB, D), lambda i: (i, 0)),
            pl.BlockSpec((1, D), lambda i: (0, 0)),
            pl.BlockSpec((1, D), lambda i: (0, 0)),
            pl.BlockSpec((1, D), lambda i: (0, 0)),
            pl.BlockSpec((1, D), lambda i: (0, 0)),
            pl.BlockSpec((D, 1), lambda i: (0, 0)),
            pl.BlockSpec((1, 1), lambda i: (0, 0)),
        ],
        out_specs=pl.BlockSpec((_SB, 1), lambda i: (i, 0)),
        out_shape=jax.ShapeDtypeStruct((PPAD, 1), jnp.float32),
    )(h, hsum, hsq, g, b, W2, b2)


def kernel(node_type_ids, edge_index, edge_weight, pairs, degrees, node_emb,
           type_emb, W_in, b_in, W_r1, b_r1, g_r1, be_r1, W_r2, b_r2, g_r2,
           be_r2, W_out, b_out, W_l1, b_l1, bn_g, bn_b, W_l2, b_l2):
    src = edge_index[1].astype(jnp.int32)
    dst = edge_index[0].astype(jnp.int32)
    ids2 = node_type_ids.astype(jnp.int32).reshape(N, 1)

    x = _tc_encode(ids2, node_emb, type_emb)
    sl, dl, wl, cnt = _sc_partition(src, dst, edge_weight)
    p = _sc_spmm_seq(x, sl, dl, wl, cnt)
    x = _tc_dense_relu(p, W_in, b_in.reshape(1, D))
    p = _sc_spmm_seq(x, sl, dl, wl, cnt)
    x = _tc_dense_ln_res(p, x, W_r1, b_r1.reshape(1, D), g_r1.reshape(1, D),
                         be_r1.reshape(1, D))
    p = _sc_spmm_seq(x, sl, dl, wl, cnt)
    x = _tc_dense_ln_res(p, x, W_r2, b_r2.reshape(1, D), g_r2.reshape(1, D),
                         be_r2.reshape(1, D))
    p = _sc_spmm_seq(x, sl, dl, wl, cnt)
    z = _tc_dense_lin(p, W_out, b_out.reshape(1, D))

    pp = jnp.pad(pairs.astype(jnp.int32), ((0, 0), (0, PPAD - P)))
    sz, dz, sd, dd = _sc_pair_gather(z, pp[0], pp[1],
                                     degrees.astype(jnp.int32))

    h, hsum, hsq = _tc_score1(
        sz, dz, sd.reshape(PPAD, 1), dd.reshape(PPAD, 1),
        W_l1[0:D], W_l1[D:2 * D], W_l1[2 * D:3 * D],
        W_l1[3 * D:3 * D + 1], W_l1[3 * D + 1:3 * D + 2], b_l1.reshape(1, D))
    out2 = _tc_score2(h, hsum, hsq, bn_g.reshape(1, D), bn_b.reshape(1, D),
                      W_l2, b_l2.reshape(1, 1))
    return out2[:P, 0]
